# baseline (speedup 1.0000x reference)
import functools

import jax
import jax.numpy as jnp
from jax import lax
from jax.experimental import pallas as pl
from jax.experimental.pallas import tpu as pltpu
from jax.experimental.pallas import tpu_sc as plsc

N = 100000
D = 128
G = 64
NC = 2
NS = 16
NW = NC * NS
L = 16
NJ = D // L

TC_ROWS = 40960
RB = 1024
SC_BASE = TC_ROWS
SC_SPAN = N - TC_ROWS
STRIDE = SC_SPAN // NW
WIN = 2048
CHUNK = 256
NCHUNK = WIN // CHUNK
GRP = 16
NGRP = CHUNK // GRP
IDS_PAD = WIN + 16

_mesh = plsc.VectorSubcoreMesh(core_axis_name="c", subcore_axis_name="s")
_params = pltpu.CompilerParams(use_tc_tiling_on_sc=False,
                               needs_layout_passes=False)

NEG_INF = float("-inf")


def _lane(vec, r):
  return jnp.max(jnp.where(lax.iota(jnp.int32, L) == r, vec, 0))


def _neg_inf_vec():
  return jnp.full((L,), NEG_INF, jnp.float32)


@functools.partial(
    pl.kernel,
    out_type=jax.ShapeDtypeStruct((NW, G, D), jnp.float32),
    mesh=_mesh,
    scratch_types=[
        pltpu.VMEM((IDS_PAD,), jnp.int32),
        pltpu.VMEM((2, CHUNK * D), jnp.float32),
        pltpu.VMEM((G, D), jnp.float32),
        pltpu.SemaphoreType.DMA,
        pltpu.SemaphoreType.DMA,
    ],
    compiler_params=_params,
)
def _partials(x_hbm, ids_hbm, part_hbm, ids_v, xbuf, acc, sem0, sem1):
  wid = lax.axis_index("s") * NC + lax.axis_index("c")
  start8 = SC_BASE + jnp.minimum((wid * STRIDE // 8) * 8, SC_SPAN - WIN)

  ids_dma = pltpu.async_copy(
      ids_hbm.at[pl.ds(start8, WIN)], ids_v.at[pl.ds(0, WIN)], sem0)

  def init_g(g, _):
    for j in range(NJ):
      acc[g, pl.ds(j * L, L)] = _neg_inf_vec()
    return 0
  lax.fori_loop(0, G, init_g, 0)

  pltpu.async_copy(x_hbm.at[pl.ds(start8 * D, CHUNK * D)], xbuf.at[0], sem1)
  ids_dma.wait()

  sems = (sem1, sem0)

  def do_chunk(c, buf_idx, carry):
    @pl.when(c + 1 < NCHUNK)
    def _():
      pltpu.async_copy(
          x_hbm.at[pl.ds((start8 + (c + 1) * CHUNK) * D, CHUNK * D)],
          xbuf.at[1 - buf_idx], sems[1 - buf_idx])

    def do_group(g, carry):
      accv = list(carry[:NJ])
      cur_seg = carry[NJ]
      base = (c * NGRP + g) * GRP
      segs = jnp.full((L,), cur_seg, jnp.int32)
      vecs = [ids_v[pl.ds(base + v * L, L)] for v in range(GRP // L)]
      same = jnp.all(jnp.equal(vecs[0], segs))
      for v in vecs[1:]:
        same = jnp.logical_and(same, jnp.all(jnp.equal(v, segs)))

      def fast(accv, cur_seg):
        out = list(accv)
        for r in range(GRP):
          for j in range(NJ):
            out[j] = jnp.maximum(
                out[j], xbuf[buf_idx, pl.ds((g * GRP + r) * D + j * L, L)])
        return tuple(out) + (cur_seg,)

      def slow(accv, cur_seg):
        for j in range(NJ):
          sl = pl.ds(j * L, L)
          acc[cur_seg, sl] = jnp.maximum(acc[cur_seg, sl], accv[j])
        def rmw_row(r, _):
          seg = _lane(vecs[0], r)
          for j in range(NJ):
            sl = pl.ds(j * L, L)
            acc[seg, sl] = jnp.maximum(
                acc[seg, sl],
                xbuf[buf_idx, pl.ds((g * GRP + r) * D + j * L, L)])
          return 0
        lax.fori_loop(0, GRP, rmw_row, 0)
        return (tuple(_neg_inf_vec() for _ in range(NJ))
                + (_lane(vecs[-1], L - 1),))

      return lax.cond(same, fast, slow, accv, cur_seg)

    return lax.fori_loop(0, NGRP, do_group, carry)

  carry = tuple(_neg_inf_vec() for _ in range(NJ)) + (jnp.int32(0),)

  def chunk_pair(p, carry):
    for b in range(2):
      pltpu.make_async_copy(x_hbm.at[pl.ds(0, CHUNK * D)], xbuf.at[b],
                            sems[b]).wait()
      carry = do_chunk(p * 2 + b, b, carry)
    return carry

  carry = lax.fori_loop(0, NCHUNK // 2, chunk_pair, carry)

  accv = carry[:NJ]
  cur_seg = carry[NJ]
  for j in range(NJ):
    sl = pl.ds(j * L, L)
    acc[cur_seg, sl] = jnp.maximum(acc[cur_seg, sl], accv[j])

  pltpu.sync_copy(acc, part_hbm.at[wid])


def _tc_partial_body(ids_ref, x_ref, out_ref):
  @pl.when(pl.program_id(0) == 0)
  def _():
    out_ref[...] = jnp.full((G, D), NEG_INF, jnp.float32)

  ids = ids_ref[...]
  lo = jnp.min(ids)
  hi = jnp.max(ids)

  single = lo == hi

  def seg_body(b, _):
    m = lax.cond(
        single,
        lambda: jnp.max(x_ref[...], axis=0, keepdims=True),
        lambda: jnp.max(jnp.where(ids == b, x_ref[...], NEG_INF), axis=0,
                        keepdims=True))
    out_ref[pl.ds(b, 1), :] = jnp.maximum(out_ref[pl.ds(b, 1), :], m)
    return 0

  lax.fori_loop(lo, hi + 1, seg_body, 0)


_tc_partial = pl.pallas_call(
    _tc_partial_body,
    grid=(TC_ROWS // RB,),
    in_specs=[
        pl.BlockSpec((RB, 1), lambda i: (i, 0)),
        pl.BlockSpec((RB, D), lambda i: (i, 0)),
    ],
    out_specs=pl.BlockSpec((G, D), lambda i: (0, 0)),
    out_shape=jax.ShapeDtypeStruct((G, D), jnp.float32),
)


def _merge_body(part_ref, tcp_ref, out_ref):
  out_ref[...] = jnp.maximum(jnp.max(part_ref[...], axis=0), tcp_ref[...])


_merge = pl.pallas_call(
    _merge_body,
    out_shape=jax.ShapeDtypeStruct((G, D), jnp.float32),
)


def kernel(x, batch):
  ids = batch.astype(jnp.int32)
  part = _partials(x.reshape(N * D), ids)
  tcp = _tc_partial(ids[:TC_ROWS].reshape(TC_ROWS, 1), x)
  return _merge(part, tcp)


# speedup vs baseline: 1.3142x; 1.3142x over previous
import functools

import jax
import jax.numpy as jnp
from jax import lax
from jax.experimental import pallas as pl
from jax.experimental.pallas import tpu as pltpu
from jax.experimental.pallas import tpu_sc as plsc

N = 100000
D = 128
G = 64
NC = 2
NS = 16
NW = NC * NS
L = 16
NJ = D // L

RPW = N // NW
WIN = 3136
CHUNK = 224
NCHUNK = WIN // CHUNK
GRP = 16
NGRP = CHUNK // GRP
IDS_PAD = WIN + 16

_mesh = plsc.VectorSubcoreMesh(core_axis_name="c", subcore_axis_name="s")
_params = pltpu.CompilerParams(use_tc_tiling_on_sc=False,
                               needs_layout_passes=False)

NEG_INF = float("-inf")


def _lane(vec, r):
  return jnp.max(jnp.where(lax.iota(jnp.int32, L) == r, vec, 0))


def _neg_inf_vec():
  return jnp.full((L,), NEG_INF, jnp.float32)


@functools.partial(
    pl.kernel,
    out_type=jax.ShapeDtypeStruct((NW, G, D), jnp.float32),
    mesh=_mesh,
    scratch_types=[
        pltpu.VMEM((IDS_PAD,), jnp.int32),
        pltpu.VMEM((2, CHUNK, D), jnp.float32),
        pltpu.VMEM((G, D), jnp.float32),
        pltpu.SemaphoreType.DMA,
        pltpu.SemaphoreType.DMA,
    ],
    compiler_params=_params,
)
def _partials(x_hbm, ids_hbm, part_hbm, ids_v, xbuf, acc, sem0, sem1):
  wid = lax.axis_index("s") * NC + lax.axis_index("c")
  row0 = wid * RPW
  start8 = jnp.minimum((row0 // 8) * 8, N - WIN)

  ids_dma = pltpu.async_copy(
      ids_hbm.at[pl.ds(start8, WIN)], ids_v.at[pl.ds(0, WIN)], sem0)

  def init_g(g, _):
    for j in range(NJ):
      acc[g, pl.ds(j * L, L)] = _neg_inf_vec()
    return 0
  lax.fori_loop(0, G, init_g, 0)

  pltpu.async_copy(x_hbm.at[pl.ds(start8, CHUNK)], xbuf.at[0], sem1)
  ids_dma.wait()

  sems = (sem1, sem0)

  def do_chunk(c, buf_idx, carry):
    @pl.when(c + 1 < NCHUNK)
    def _():
      pltpu.async_copy(x_hbm.at[pl.ds(start8 + (c + 1) * CHUNK, CHUNK)],
                       xbuf.at[1 - buf_idx], sems[1 - buf_idx])

    def do_group(g, carry):
      accv = list(carry[:NJ])
      cur_seg = carry[NJ]
      base = (c * NGRP + g) * GRP
      segs = jnp.full((L,), cur_seg, jnp.int32)
      vecs = [ids_v[pl.ds(base + v * L, L)] for v in range(GRP // L)]
      same = jnp.all(jnp.equal(vecs[0], segs))
      for v in vecs[1:]:
        same = jnp.logical_and(same, jnp.all(jnp.equal(v, segs)))

      def fast(accv, cur_seg):
        out = list(accv)
        for r in range(GRP):
          for j in range(NJ):
            out[j] = jnp.maximum(out[j], xbuf[buf_idx, g * GRP + r,
                                              pl.ds(j * L, L)])
        return tuple(out) + (cur_seg,)

      def slow(accv, cur_seg):
        for j in range(NJ):
          sl = pl.ds(j * L, L)
          acc[cur_seg, sl] = jnp.maximum(acc[cur_seg, sl], accv[j])
        def rmw_row(r, _):
          seg = _lane(vecs[0], r)
          for j in range(NJ):
            sl = pl.ds(j * L, L)
            acc[seg, sl] = jnp.maximum(acc[seg, sl],
                                       xbuf[buf_idx, g * GRP + r, sl])
          return 0
        lax.fori_loop(0, GRP, rmw_row, 0)
        return (tuple(_neg_inf_vec() for _ in range(NJ))
                + (_lane(vecs[-1], L - 1),))

      return lax.cond(same, fast, slow, accv, cur_seg)

    return lax.fori_loop(0, NGRP, do_group, carry)

  carry = tuple(_neg_inf_vec() for _ in range(NJ)) + (jnp.int32(0),)

  def chunk_pair(p, carry):
    for b in range(2):
      pltpu.make_async_copy(x_hbm.at[pl.ds(0, CHUNK)], xbuf.at[b],
                            sems[b]).wait()
      carry = do_chunk(p * 2 + b, b, carry)
    return carry

  carry = lax.fori_loop(0, NCHUNK // 2, chunk_pair, carry)

  accv = carry[:NJ]
  cur_seg = carry[NJ]
  for j in range(NJ):
    sl = pl.ds(j * L, L)
    acc[cur_seg, sl] = jnp.maximum(acc[cur_seg, sl], accv[j])

  pltpu.sync_copy(acc, part_hbm.at[wid])


def _merge_body(part_ref, out_ref):
  out_ref[...] = jnp.max(part_ref[...], axis=0)


_merge = pl.pallas_call(
    _merge_body,
    out_shape=jax.ShapeDtypeStruct((G, D), jnp.float32),
)


def kernel(x, batch):
  ids = batch.astype(jnp.int32)
  part = _partials(x, ids)
  return _merge(part)


# speedup vs baseline: 1.3195x; 1.0041x over previous
import functools

import jax
import jax.numpy as jnp
from jax import lax
from jax.experimental import pallas as pl
from jax.experimental.pallas import tpu as pltpu
from jax.experimental.pallas import tpu_sc as plsc

N = 100000
D = 128
G = 64
NC = 2
NS = 16
NW = NC * NS
L = 16
NJ = D // L

RPW = N // NW
WIN = 3136
CHUNK = 224
NCHUNK = WIN // CHUNK
GRP = 16
NGRP = CHUNK // GRP
IDS_PAD = WIN + 16

_mesh = plsc.VectorSubcoreMesh(core_axis_name="c", subcore_axis_name="s")
_params = pltpu.CompilerParams(use_tc_tiling_on_sc=False,
                               needs_layout_passes=False)

NEG_INF = float("-inf")


def _lane(vec, r):
  return jnp.max(jnp.where(lax.iota(jnp.int32, L) == r, vec, 0))


def _neg_inf_vec():
  return jnp.full((L,), NEG_INF, jnp.float32)


@functools.partial(
    pl.kernel,
    out_type=jax.ShapeDtypeStruct((NW, G, D), jnp.float32),
    mesh=_mesh,
    scratch_types=[
        pltpu.VMEM((IDS_PAD,), jnp.int32),
        pltpu.VMEM((2, CHUNK, D), jnp.float32),
        pltpu.VMEM((G, D), jnp.float32),
        pltpu.SemaphoreType.DMA,
        pltpu.SemaphoreType.DMA,
    ],
    compiler_params=_params,
)
def _partials(x_hbm, ids_hbm, part_hbm, ids_v, xbuf, acc, sem0, sem1):
  wid = lax.axis_index("s") * NC + lax.axis_index("c")
  row0 = wid * RPW
  start8 = jnp.minimum((row0 // 8) * 8, N - WIN)

  ids_dma = pltpu.async_copy(
      ids_hbm.at[pl.ds(start8, WIN)], ids_v.at[pl.ds(0, WIN)], sem0)

  def init_g(g, _):
    for j in range(NJ):
      acc[g, pl.ds(j * L, L)] = _neg_inf_vec()
    return 0
  lax.fori_loop(0, G, init_g, 0)

  pltpu.async_copy(x_hbm.at[pl.ds(start8, CHUNK)], xbuf.at[0], sem1)
  ids_dma.wait()

  sems = (sem1, sem0)

  def do_chunk(c, buf_idx, carry):
    @pl.when(c + 1 < NCHUNK)
    def _():
      pltpu.async_copy(x_hbm.at[pl.ds(start8 + (c + 1) * CHUNK, CHUNK)],
                       xbuf.at[1 - buf_idx], sems[1 - buf_idx])

    def group_accumulate(g, accv):
      out = list(accv)
      for r in range(GRP):
        for j in range(NJ):
          out[j] = jnp.maximum(out[j], xbuf[buf_idx, g * GRP + r,
                                            pl.ds(j * L, L)])
      return tuple(out)

    def do_group(g, carry):
      accv = list(carry[:NJ])
      cur_seg = carry[NJ]
      base = (c * NGRP + g) * GRP
      segs = jnp.full((L,), cur_seg, jnp.int32)
      vec = ids_v[pl.ds(base, L)]
      same = jnp.all(jnp.equal(vec, segs))

      def fast(accv, cur_seg):
        return group_accumulate(g, accv) + (cur_seg,)

      def slow(accv, cur_seg):
        for j in range(NJ):
          sl = pl.ds(j * L, L)
          acc[cur_seg, sl] = jnp.maximum(acc[cur_seg, sl], accv[j])
        def rmw_row(r, _):
          seg = _lane(vec, r)
          for j in range(NJ):
            sl = pl.ds(j * L, L)
            acc[seg, sl] = jnp.maximum(acc[seg, sl],
                                       xbuf[buf_idx, g * GRP + r, sl])
          return 0
        lax.fori_loop(0, GRP, rmw_row, 0)
        return (tuple(_neg_inf_vec() for _ in range(NJ))
                + (_lane(vec, L - 1),))

      return lax.cond(same, fast, slow, accv, cur_seg)

    cur_seg0 = carry[NJ]
    segs0 = jnp.full((L,), cur_seg0, jnp.int32)
    chunk_same = jnp.bool_(True)
    for k in range(NGRP):
      v = ids_v[pl.ds((c * NGRP + k) * GRP, L)]
      chunk_same = jnp.logical_and(chunk_same,
                                   jnp.all(jnp.equal(v, segs0)))

    def fast_chunk(carry):
      def fg(g, cy):
        return group_accumulate(g, cy[:NJ]) + (cy[NJ],)
      return lax.fori_loop(0, NGRP, fg, carry)

    def slow_chunk(carry):
      return lax.fori_loop(0, NGRP, do_group, carry)

    return lax.cond(chunk_same, fast_chunk, slow_chunk, carry)

  carry = tuple(_neg_inf_vec() for _ in range(NJ)) + (jnp.int32(0),)

  def chunk_pair(p, carry):
    for b in range(2):
      pltpu.make_async_copy(x_hbm.at[pl.ds(0, CHUNK)], xbuf.at[b],
                            sems[b]).wait()
      carry = do_chunk(p * 2 + b, b, carry)
    return carry

  carry = lax.fori_loop(0, NCHUNK // 2, chunk_pair, carry)

  accv = carry[:NJ]
  cur_seg = carry[NJ]
  for j in range(NJ):
    sl = pl.ds(j * L, L)
    acc[cur_seg, sl] = jnp.maximum(acc[cur_seg, sl], accv[j])

  pltpu.sync_copy(acc, part_hbm.at[wid])


def _merge_body(part_ref, out_ref):
  out_ref[...] = jnp.max(part_ref[...], axis=0)


_merge = pl.pallas_call(
    _merge_body,
    out_shape=jax.ShapeDtypeStruct((G, D), jnp.float32),
)


def kernel(x, batch):
  ids = batch.astype(jnp.int32)
  part = _partials(x, ids)
  return _merge(part)
